# baseline (speedup 1.0000x reference)
import functools

import jax
import jax.numpy as jnp
from jax import lax
from jax.experimental import pallas as pl
from jax.experimental.pallas import tpu as pltpu
from jax.experimental.pallas import tpu_sc as plsc

N = 65536
K = 8192
D = 32
TN = 512
NB = N // TN

_COMMITMENT_COST = 0.25
_DIVERGENCE_COST = 1.0


_KCH = K // 128
_NG = TN // 8


def _argmin_body(x_ref, x2_ref, w2x_ref, w2_ref, idx_ref, loss_ref,
                 m_buf, jw_ref):
    i = pl.program_id(0)

    @pl.when(i < NB)
    def _dot():
        sl = lax.rem(i, 2)
        m_buf[pl.ds(sl * TN, TN), :] = lax.dot_general(
            x_ref[...], w2x_ref[...], (((1,), (1,)), ((), ())),
            preferred_element_type=jnp.float32)

    @pl.when(i > 0)
    def _scan():
        base = lax.rem(i + 1, 2) * TN
        lane = lax.broadcasted_iota(jnp.int32, (8, 128), 1).astype(
            jnp.float32)

        def group(g, loss_acc):
            x2g = x2_ref[pl.ds(g * 8, 8), :]
            run_v = jnp.full((8, 128), jnp.inf, jnp.float32)
            run_c = jnp.zeros((8, 128), jnp.float32)
            for cc in range(_KCH):
                mm = m_buf[pl.ds(base + g * 8, 8), cc * 128:(cc + 1) * 128]
                w2c = w2_ref[:, cc * 128:(cc + 1) * 128]
                d2c = (x2g - mm) + w2c
                cf = jnp.maximum(d2c, 1e-30)
                dc = cf * lax.rsqrt(cf)
                better = dc < run_v
                run_v = jnp.minimum(run_v, dc)
                run_c = jnp.where(better, jnp.float32(cc), run_c)
            vrow = jnp.min(run_v, axis=1, keepdims=True)
            jl = run_c * 128.0 + lane
            jwin = jnp.min(jnp.where(run_v == vrow, jl, jnp.float32(K)),
                           axis=1, keepdims=True)
            jw_ref[pl.ds(g * 8, 8), :] = jwin
            return loss_acc + jnp.sum(vrow * vrow)

        lt = lax.fori_loop(0, _NG, group, jnp.float32(0.0))
        idx_ref[...] = jw_ref[...].astype(jnp.int32).reshape(1, 1, TN)

        @pl.when(i == 1)
        def _():
            loss_ref[...] = jnp.zeros_like(loss_ref)

        loss_ref[...] += jnp.reshape(lt, (1, 1))


_argmin_call = pl.pallas_call(
    _argmin_body,
    grid=(NB + 1,),
    in_specs=[
        pl.BlockSpec((TN, D), lambda i: (jnp.minimum(i, NB - 1), 0)),
        pl.BlockSpec((TN, 1), lambda i: (jnp.maximum(i - 1, 0), 0)),
        pl.BlockSpec((K, D), lambda i: (0, 0)),
        pl.BlockSpec((1, K), lambda i: (0, 0)),
    ],
    out_specs=[
        pl.BlockSpec((1, 1, TN), lambda i: (jnp.maximum(i - 1, 0), 0, 0)),
        pl.BlockSpec((1, 1), lambda i: (0, 0)),
    ],
    out_shape=[
        jax.ShapeDtypeStruct((NB, 1, TN), jnp.int32),
        jax.ShapeDtypeStruct((1, 1), jnp.float32),
    ],
    scratch_shapes=[
        pltpu.VMEM((2 * TN, K), jnp.float32),
        pltpu.VMEM((TN, 1), jnp.float32),
    ],
)

_NC = 2
_NS = 16
_NW = _NC * _NS
_BPW = N // _NW
_CH = 128
_NCH = _BPW // _CH


@functools.cache
def _sc_gather_call():
    @functools.partial(
        pl.kernel,
        out_type=jax.ShapeDtypeStruct((N, D), jnp.float32),
        mesh=plsc.VectorSubcoreMesh(core_axis_name="c", subcore_axis_name="s"),
        scratch_types=[
            pltpu.VMEM((_NCH, _CH), jnp.int32),
            pltpu.VMEM((_BPW, D), jnp.float32),
            pltpu.SemaphoreType.DMA,
        ],
        compiler_params=pltpu.CompilerParams(use_tc_tiling_on_sc=False),
    )
    def _sc_gather(idx_hbm, w_hbm, out_hbm, idx_v, rows_v, sem):
        wid = lax.axis_index("s") * _NC + lax.axis_index("c")
        base = wid * _BPW
        pltpu.sync_copy(idx_hbm.at[wid], idx_v)
        copies = []
        for j in range(_NCH):
            copies.append(pltpu.async_copy(
                w_hbm.at[idx_v.at[j]], rows_v.at[pl.ds(j * _CH, _CH)], sem))
        for c in copies:
            c.wait()
        pltpu.sync_copy(rows_v, out_hbm.at[pl.ds(base, _BPW)])

    return _sc_gather


def kernel(inputs, W):
    x2 = jnp.sum(inputs ** 2, axis=1, keepdims=True)
    w2 = jnp.sum(W ** 2, axis=1)[None, :]
    idx3, losssum = _argmin_call(inputs, x2, W + W, w2)
    idx_r = idx3.reshape(_NW, _NCH, _CH)
    quantized = _sc_gather_call()(idx_r, W)
    m = losssum[0, 0] / jnp.float32(N * D)
    loss = m * _DIVERGENCE_COST + _COMMITMENT_COST * m
    return (quantized, loss)


# speedup vs baseline: 3.5004x; 3.5004x over previous
import functools

import jax
import jax.numpy as jnp
from jax import lax
from jax.experimental import pallas as pl
from jax.experimental.pallas import tpu as pltpu
from jax.experimental.pallas import tpu_sc as plsc

N = 65536
K = 8192
D = 32
TN = 1024
NB = N // TN

_COMMITMENT_COST = 0.25
_DIVERGENCE_COST = 1.0


def _argmin_body(x_ref, x2_ref, w2x_ref, w2_ref, idx_ref, loss_ref):
    m2 = lax.dot_general(
        x_ref[...], w2x_ref[...], (((1,), (1,)), ((), ())),
        preferred_element_type=jnp.float32)
    d2 = (x2_ref[...] - m2) + w2_ref[...]
    c = jnp.maximum(d2, 1e-30)
    dist = c * lax.rsqrt(c)
    minval = jnp.min(dist, axis=1, keepdims=True)
    fiota = lax.broadcasted_iota(jnp.int32, (TN, K), 1).astype(jnp.float32)
    idx_f = jnp.min(jnp.where(dist == minval, fiota, jnp.float32(K)), axis=1)
    idx_ref[...] = idx_f.astype(jnp.int32).reshape(1, 1, TN)

    @pl.when(pl.program_id(0) == 0)
    def _():
        loss_ref[...] = jnp.zeros_like(loss_ref)

    loss_ref[...] += jnp.sum(minval * minval, keepdims=True)


_argmin_call = pl.pallas_call(
    _argmin_body,
    grid=(NB,),
    in_specs=[
        pl.BlockSpec((TN, D), lambda i: (i, 0)),
        pl.BlockSpec((TN, 1), lambda i: (i, 0)),
        pl.BlockSpec((K, D), lambda i: (0, 0)),
        pl.BlockSpec((1, K), lambda i: (0, 0)),
    ],
    out_specs=[
        pl.BlockSpec((1, 1, TN), lambda i: (i, 0, 0)),
        pl.BlockSpec((1, 1), lambda i: (0, 0)),
    ],
    out_shape=[
        jax.ShapeDtypeStruct((NB, 1, TN), jnp.int32),
        jax.ShapeDtypeStruct((1, 1), jnp.float32),
    ],
)

_NC = 2
_NS = 16
_NW = _NC * _NS
_BPW = N // _NW
_CH = 128
_NCH = _BPW // _CH


@functools.cache
def _sc_gather_call():
    @functools.partial(
        pl.kernel,
        out_type=jax.ShapeDtypeStruct((N, D), jnp.float32),
        mesh=plsc.VectorSubcoreMesh(core_axis_name="c", subcore_axis_name="s"),
        scratch_types=[
            pltpu.VMEM((_NCH, _CH), jnp.int32),
            pltpu.VMEM((_BPW, D), jnp.float32),
            pltpu.SemaphoreType.DMA,
        ],
        compiler_params=pltpu.CompilerParams(use_tc_tiling_on_sc=False),
    )
    def _sc_gather(idx_hbm, w_hbm, out_hbm, idx_v, rows_v, sem):
        wid = lax.axis_index("s") * _NC + lax.axis_index("c")
        base = wid * _BPW
        pltpu.sync_copy(idx_hbm.at[wid], idx_v)
        copies = []
        for j in range(_NCH):
            copies.append(pltpu.async_copy(
                w_hbm.at[idx_v.at[j]], rows_v.at[pl.ds(j * _CH, _CH)], sem))
        for c in copies:
            c.wait()
        pltpu.sync_copy(rows_v, out_hbm.at[pl.ds(base, _BPW)])

    return _sc_gather


def kernel(inputs, W):
    x2 = jnp.sum(inputs ** 2, axis=1, keepdims=True)
    w2 = jnp.sum(W ** 2, axis=1)[None, :]
    idx3, losssum = _argmin_call(inputs, x2, W + W, w2)
    idx_r = idx3.reshape(_NW, _NCH, _CH)
    quantized = _sc_gather_call()(idx_r, W)
    m = losssum[0, 0] / jnp.float32(N * D)
    loss = m * _DIVERGENCE_COST + _COMMITMENT_COST * m
    return (quantized, loss)


# speedup vs baseline: 3.5715x; 1.0203x over previous
import functools

import jax
import jax.numpy as jnp
from jax import lax
from jax.experimental import pallas as pl
from jax.experimental.pallas import tpu as pltpu
from jax.experimental.pallas import tpu_sc as plsc

N = 65536
K = 8192
D = 32
TN = 1024
NB = N // TN

_COMMITMENT_COST = 0.25
_DIVERGENCE_COST = 1.0

_CH = 128
_IDXR = N // _CH


def _argmin_body(x_ref, x2_ref, w2x_ref, w2_ref, idx_ref, loss_ref):
    m2 = lax.dot_general(
        x_ref[...], w2x_ref[...], (((1,), (1,)), ((), ())),
        preferred_element_type=jnp.float32)
    d2 = (x2_ref[...] - m2) + w2_ref[...]
    c = jnp.maximum(d2, 1e-30)
    dist = c * lax.rsqrt(c)
    minval = jnp.min(dist, axis=1, keepdims=True)
    fiota = lax.broadcasted_iota(jnp.int32, (TN, K), 1).astype(jnp.float32)
    idx_f = jnp.min(jnp.where(dist == minval, fiota, jnp.float32(K)), axis=1)
    idx_ref[...] = idx_f.astype(jnp.int32).reshape(TN // _CH, _CH)

    @pl.when(pl.program_id(0) == 0)
    def _():
        loss_ref[...] = jnp.zeros_like(loss_ref)

    loss_ref[...] += jnp.sum(minval * minval, keepdims=True)


_argmin_call = pl.pallas_call(
    _argmin_body,
    grid=(NB,),
    in_specs=[
        pl.BlockSpec((TN, D), lambda i: (i, 0)),
        pl.BlockSpec((TN, 1), lambda i: (i, 0)),
        pl.BlockSpec((K, D), lambda i: (0, 0)),
        pl.BlockSpec((1, K), lambda i: (0, 0)),
    ],
    out_specs=[
        pl.BlockSpec((TN // _CH, _CH), lambda i: (i, 0)),
        pl.BlockSpec((1, 1), lambda i: (0, 0)),
    ],
    out_shape=[
        jax.ShapeDtypeStruct((_IDXR, _CH), jnp.int32),
        jax.ShapeDtypeStruct((1, 1), jnp.float32),
    ],
)

_NC = 2
_NS = 16
_NW = _NC * _NS
_BPW = N // _NW
_NCH = _BPW // _CH


@functools.cache
def _sc_gather_call():
    @functools.partial(
        pl.kernel,
        out_type=jax.ShapeDtypeStruct((N, D), jnp.float32),
        mesh=plsc.VectorSubcoreMesh(core_axis_name="c", subcore_axis_name="s"),
        scratch_types=[
            pltpu.VMEM((_NCH, _CH), jnp.int32),
            pltpu.VMEM((_BPW, D), jnp.float32),
            pltpu.SemaphoreType.DMA,
        ],
        compiler_params=pltpu.CompilerParams(use_tc_tiling_on_sc=False),
    )
    def _sc_gather(idx_hbm, w_hbm, out_hbm, idx_v, rows_v, sem):
        wid = lax.axis_index("s") * _NC + lax.axis_index("c")
        base = wid * _BPW
        pltpu.sync_copy(idx_hbm.at[pl.ds(wid * _NCH, _NCH)], idx_v)
        copies = []
        for j in range(_NCH):
            copies.append(pltpu.async_copy(
                w_hbm.at[idx_v.at[j]], rows_v.at[pl.ds(j * _CH, _CH)], sem))
        for c in copies:
            c.wait()
        pltpu.sync_copy(rows_v, out_hbm.at[pl.ds(base, _BPW)])

    return _sc_gather


def kernel(inputs, W):
    x2 = jnp.sum(inputs ** 2, axis=1, keepdims=True)
    w2 = jnp.sum(W ** 2, axis=1)[None, :]
    idx2, losssum = _argmin_call(inputs, x2, W + W, w2)
    quantized = _sc_gather_call()(idx2, W)
    m = losssum[0, 0] / jnp.float32(N * D)
    loss = m * _DIVERGENCE_COST + _COMMITMENT_COST * m
    return (quantized, loss)


# speedup vs baseline: 3.5767x; 1.0015x over previous
import functools

import jax
import jax.numpy as jnp
from jax import lax
from jax.experimental import pallas as pl
from jax.experimental.pallas import tpu as pltpu
from jax.experimental.pallas import tpu_sc as plsc

N = 65536
K = 8192
D = 32
TN = 1024
NB = N // TN

_COMMITMENT_COST = 0.25
_DIVERGENCE_COST = 1.0

_CH = 128
_IDXR = N // _CH


def _argmin_body(x_ref, x2_ref, w2x_ref, w2_ref, idx_ref, loss_ref):
    m2 = lax.dot_general(
        x_ref[...], w2x_ref[...], (((1,), (1,)), ((), ())),
        preferred_element_type=jnp.float32)
    d2 = (x2_ref[...] - m2) + w2_ref[...]
    c = jnp.maximum(d2, 1e-30)
    dist = c * lax.rsqrt(c)
    minval = jnp.min(dist, axis=1, keepdims=True)
    fiota = lax.broadcasted_iota(jnp.int32, (TN, K), 1).astype(jnp.float32)
    idx_f = jnp.min(jnp.where(dist == minval, fiota, jnp.float32(K)), axis=1)
    idx_ref[...] = idx_f.astype(jnp.int32).reshape(TN // _CH, _CH)

    @pl.when(pl.program_id(0) == 0)
    def _():
        loss_ref[...] = jnp.zeros_like(loss_ref)

    loss_ref[...] += jnp.sum(minval * minval, keepdims=True)


_argmin_call = pl.pallas_call(
    _argmin_body,
    grid=(NB,),
    in_specs=[
        pl.BlockSpec((TN, D), lambda i: (i, 0)),
        pl.BlockSpec((TN, 1), lambda i: (i, 0)),
        pl.BlockSpec((K, D), lambda i: (0, 0)),
        pl.BlockSpec((1, K), lambda i: (0, 0)),
    ],
    out_specs=[
        pl.BlockSpec((TN // _CH, _CH), lambda i: (i, 0)),
        pl.BlockSpec((1, 1), lambda i: (0, 0)),
    ],
    out_shape=[
        jax.ShapeDtypeStruct((_IDXR, _CH), jnp.int32),
        jax.ShapeDtypeStruct((1, 1), jnp.float32),
    ],
    compiler_params=pltpu.CompilerParams(
        allow_input_fusion=(False, True, True, True)),
)

_NC = 2
_NS = 16
_NW = _NC * _NS
_BPW = N // _NW
_NCH = _BPW // _CH


@functools.cache
def _sc_gather_call():
    @functools.partial(
        pl.kernel,
        out_type=jax.ShapeDtypeStruct((N, D), jnp.float32),
        mesh=plsc.VectorSubcoreMesh(core_axis_name="c", subcore_axis_name="s"),
        scratch_types=[
            pltpu.VMEM((_NCH, _CH), jnp.int32),
            pltpu.VMEM((_BPW, D), jnp.float32),
            pltpu.SemaphoreType.DMA,
        ],
        compiler_params=pltpu.CompilerParams(use_tc_tiling_on_sc=False),
    )
    def _sc_gather(idx_hbm, w_hbm, out_hbm, idx_v, rows_v, sem):
        wid = lax.axis_index("s") * _NC + lax.axis_index("c")
        base = wid * _BPW
        pltpu.sync_copy(idx_hbm.at[pl.ds(wid * _NCH, _NCH)], idx_v)
        copies = []
        for j in range(_NCH):
            copies.append(pltpu.async_copy(
                w_hbm.at[idx_v.at[j]], rows_v.at[pl.ds(j * _CH, _CH)], sem))
        for c in copies:
            c.wait()
        pltpu.sync_copy(rows_v, out_hbm.at[pl.ds(base, _BPW)])

    return _sc_gather


def kernel(inputs, W):
    x2 = jnp.sum(inputs ** 2, axis=1, keepdims=True)
    w2 = jnp.sum(W ** 2, axis=1)[None, :]
    idx2, losssum = _argmin_call(inputs, x2, W + W, w2)
    quantized = _sc_gather_call()(idx2, W)
    m = losssum[0, 0] / jnp.float32(N * D)
    loss = m * _DIVERGENCE_COST + _COMMITMENT_COST * m
    return (quantized, loss)


# speedup vs baseline: 3.5794x; 1.0008x over previous
import functools

import jax
import jax.numpy as jnp
from jax import lax
from jax.experimental import pallas as pl
from jax.experimental.pallas import tpu as pltpu
from jax.experimental.pallas import tpu_sc as plsc

N = 65536
K = 8192
D = 32
TN = 1024
NB = N // TN

_COMMITMENT_COST = 0.25
_DIVERGENCE_COST = 1.0

_CH = 128
_IDXR = N // _CH


def _argmin_body(x_ref, x2_ref, w2x_ref, w2_ref, idx_ref, loss_ref):
    m2 = lax.dot_general(
        x_ref[...], w2x_ref[...], (((1,), (1,)), ((), ())),
        preferred_element_type=jnp.float32)
    d2 = (x2_ref[...] - m2) + w2_ref[...]
    c = jnp.maximum(d2, 1e-30)
    dist = c * lax.rsqrt(c)
    minval = jnp.min(dist, axis=1, keepdims=True)
    fiota = lax.broadcasted_iota(jnp.int32, (TN, K), 1).astype(jnp.float32)
    idx_f = jnp.min(jnp.where(dist == minval, fiota, jnp.float32(K)), axis=1)
    idx_ref[...] = idx_f.astype(jnp.int32).reshape(TN // _CH, _CH)

    @pl.when(pl.program_id(0) == 0)
    def _():
        loss_ref[...] = jnp.zeros_like(loss_ref)

    loss_ref[...] += jnp.sum(minval * minval, keepdims=True)


_argmin_call = pl.pallas_call(
    _argmin_body,
    grid=(NB,),
    in_specs=[
        pl.BlockSpec((TN, D), lambda i: (i, 0)),
        pl.BlockSpec((TN, 1), lambda i: (i, 0)),
        pl.BlockSpec((K, D), lambda i: (0, 0)),
        pl.BlockSpec((1, K), lambda i: (0, 0)),
    ],
    out_specs=[
        pl.BlockSpec((TN // _CH, _CH), lambda i: (i, 0)),
        pl.BlockSpec((1, 1), lambda i: (0, 0)),
    ],
    out_shape=[
        jax.ShapeDtypeStruct((_IDXR, _CH), jnp.int32),
        jax.ShapeDtypeStruct((1, 1), jnp.float32),
    ],
)

_NC = 2
_NS = 16
_NW = _NC * _NS
_BPW = N // _NW
_NCH = _BPW // _CH


@functools.cache
def _sc_gather_call():
    @functools.partial(
        pl.kernel,
        out_type=jax.ShapeDtypeStruct((N, D), jnp.float32),
        mesh=plsc.VectorSubcoreMesh(core_axis_name="c", subcore_axis_name="s"),
        scratch_types=[
            pltpu.VMEM((_NCH, _CH), jnp.int32),
            pltpu.VMEM((_BPW, D), jnp.float32),
            pltpu.SemaphoreType.DMA,
        ],
        compiler_params=pltpu.CompilerParams(use_tc_tiling_on_sc=False),
    )
    def _sc_gather(idx_hbm, w_hbm, out_hbm, idx_v, rows_v, sem):
        wid = lax.axis_index("s") * _NC + lax.axis_index("c")
        base = wid * _BPW
        pltpu.sync_copy(idx_hbm.at[pl.ds(wid * _NCH, _NCH)], idx_v)
        copies = []
        for j in range(_NCH):
            copies.append(pltpu.async_copy(
                w_hbm.at[idx_v.at[j]], rows_v.at[pl.ds(j * _CH, _CH)], sem))
        for c in copies:
            c.wait()
        pltpu.sync_copy(rows_v, out_hbm.at[pl.ds(base, _BPW)])

    return _sc_gather


def kernel(inputs, W):
    x2 = jnp.sum(inputs ** 2, axis=1, keepdims=True)
    w2 = jnp.sum(W ** 2, axis=1)[None, :]
    idx2, losssum = _argmin_call(inputs, x2, W + W, w2)
    quantized = _sc_gather_call()(idx2, W)
    m = losssum[0, 0] / jnp.float32(N * D)
    loss = m * _DIVERGENCE_COST + _COMMITMENT_COST * m
    return (quantized, loss)
